# Initial kernel scaffold; baseline (speedup 1.0000x reference)
import functools

import jax
import jax.numpy as jnp
from jax.experimental import pallas as pl

N_TOK = 36864
N_CODES = 1024
DIM = 64
BLK = 512


def _vq_kernel(x_ref, cb_ref, l2_ref, codes_ref, vec_ref):
    x = x_ref[...]
    cb = cb_ref[...]
    cross = jax.lax.dot_general(
        x, cb, (((1,), (1,)), ((), ())),
        preferred_element_type=jnp.float32,
        precision=jax.lax.Precision.HIGHEST,
    )
    c2 = jnp.sum(cb * cb, axis=1)
    e = c2[None, :] - 2.0 * cross
    x2 = jnp.sum(x * x, axis=1)
    l2_ref[...] = x2[:, None] + e
    codes = jnp.argmin(e, axis=1).astype(jnp.int32)
    codes_ref[...] = codes
    onehot = (codes[:, None] == jax.lax.broadcasted_iota(
        jnp.int32, (1, N_CODES), 1)).astype(jnp.float32)
    vec_ref[...] = jax.lax.dot_general(
        onehot, cb, (((1,), (0,)), ((), ())),
        preferred_element_type=jnp.float32,
        precision=jax.lax.Precision.HIGHEST,
    )


@functools.partial(jax.jit, static_argnames=())
def kernel(x, codebook):
    grid = (N_TOK // BLK,)
    l2, codes, vec = pl.pallas_call(
        _vq_kernel,
        grid=grid,
        in_specs=[
            pl.BlockSpec((BLK, DIM), lambda i: (i, 0)),
            pl.BlockSpec((N_CODES, DIM), lambda i: (0, 0)),
        ],
        out_specs=[
            pl.BlockSpec((BLK, N_CODES), lambda i: (i, 0)),
            pl.BlockSpec((BLK,), lambda i: (i,)),
            pl.BlockSpec((BLK, DIM), lambda i: (i, 0)),
        ],
        out_shape=[
            jax.ShapeDtypeStruct((N_TOK, N_CODES), jnp.float32),
            jax.ShapeDtypeStruct((N_TOK,), jnp.int32),
            jax.ShapeDtypeStruct((N_TOK, DIM), jnp.float32),
        ],
    )(x, codebook)
    return (vec, codes, l2)


# speedup vs baseline: 7.5467x; 7.5467x over previous
import functools

import jax
import jax.numpy as jnp
from jax.experimental import pallas as pl

N_TOK = 36864
N_CODES = 1024
DIM = 64
BLK = 512


def _split(a):
    hi = a.astype(jnp.bfloat16)
    lo = (a - hi.astype(jnp.float32)).astype(jnp.bfloat16)
    return hi, lo


def _mm(a, b):
    return jax.lax.dot_general(
        a, b, (((1,), (0,)), ((), ())),
        preferred_element_type=jnp.float32)


def _vq_kernel(x_ref, cbt2_ref, cb_ref, c2_ref, l2_ref, codes_ref, vec_ref):
    x = x_ref[...]
    cbt2 = cbt2_ref[...]
    xh, xl = _split(x)
    ch, cl = _split(cbt2)
    cross = _mm(xh, ch) + _mm(xh, cl) + _mm(xl, ch)
    e = c2_ref[...] + cross
    x2 = jnp.sum(x * x, axis=1, keepdims=True)
    l2_ref[...] = x2 + e
    codes = jnp.argmin(e, axis=1).astype(jnp.int32)
    codes_ref[...] = codes
    onehot = (codes[:, None] == jax.lax.broadcasted_iota(
        jnp.int32, (1, N_CODES), 1)).astype(jnp.bfloat16)
    cbh, cbl = _split(cb_ref[...])
    vec_ref[...] = _mm(onehot, cbh) + _mm(onehot, cbl)


@functools.partial(jax.jit, static_argnames=())
def kernel(x, codebook):
    cbt2 = -2.0 * codebook.T
    c2 = jnp.sum(codebook * codebook, axis=1)[None, :]

    grid = (N_TOK // BLK,)
    l2, codes, vec = pl.pallas_call(
        _vq_kernel,
        grid=grid,
        in_specs=[
            pl.BlockSpec((BLK, DIM), lambda i: (i, 0)),
            pl.BlockSpec((DIM, N_CODES), lambda i: (0, 0)),
            pl.BlockSpec((N_CODES, DIM), lambda i: (0, 0)),
            pl.BlockSpec((1, N_CODES), lambda i: (0, 0)),
        ],
        out_specs=[
            pl.BlockSpec((BLK, N_CODES), lambda i: (i, 0)),
            pl.BlockSpec((BLK,), lambda i: (i,)),
            pl.BlockSpec((BLK, DIM), lambda i: (i, 0)),
        ],
        out_shape=[
            jax.ShapeDtypeStruct((N_TOK, N_CODES), jnp.float32),
            jax.ShapeDtypeStruct((N_TOK,), jnp.int32),
            jax.ShapeDtypeStruct((N_TOK, DIM), jnp.float32),
        ],
    )(x, cbt2, codebook, c2)
    return (vec, codes, l2)


# speedup vs baseline: 9.2010x; 1.2192x over previous
import functools

import jax
import jax.numpy as jnp
from jax.experimental import pallas as pl

N_TOK = 36864
N_CODES = 1024
DIM = 64
BLK = 256


def _split(a):
    hi = a.astype(jnp.bfloat16)
    lo = (a - hi.astype(jnp.float32)).astype(jnp.bfloat16)
    return hi, lo


def _mm(a, b):
    return jax.lax.dot_general(
        a, b, (((1,), (0,)), ((), ())),
        preferred_element_type=jnp.float32)


def _vq_kernel(x_ref, cbt2_ref, cb_ref, c2_ref, l2_ref, codes_ref, vec_ref):
    x = x_ref[...]
    cbt2 = cbt2_ref[...]
    xh, xl = _split(x)
    ch, cl = _split(cbt2)
    cross = _mm(xh, ch) + _mm(xh, cl) + _mm(xl, ch)
    e = c2_ref[...] + cross
    x2 = jnp.sum(x * x, axis=1, keepdims=True)
    l2_ref[...] = x2 + e
    codes = jnp.argmin(e, axis=1).astype(jnp.int32)
    codes_ref[...] = codes
    onehot = (codes[:, None] == jax.lax.broadcasted_iota(
        jnp.int32, (1, N_CODES), 1)).astype(jnp.bfloat16)
    cbh, cbl = _split(cb_ref[...])
    vec_ref[...] = _mm(onehot, cbh) + _mm(onehot, cbl)


@functools.partial(jax.jit, static_argnames=())
def kernel(x, codebook):
    cbt2 = -2.0 * codebook.T
    c2 = jnp.sum(codebook * codebook, axis=1)[None, :]

    grid = (N_TOK // BLK,)
    l2, codes, vec = pl.pallas_call(
        _vq_kernel,
        grid=grid,
        in_specs=[
            pl.BlockSpec((BLK, DIM), lambda i: (i, 0)),
            pl.BlockSpec((DIM, N_CODES), lambda i: (0, 0)),
            pl.BlockSpec((N_CODES, DIM), lambda i: (0, 0)),
            pl.BlockSpec((1, N_CODES), lambda i: (0, 0)),
        ],
        out_specs=[
            pl.BlockSpec((BLK, N_CODES), lambda i: (i, 0)),
            pl.BlockSpec((BLK,), lambda i: (i,)),
            pl.BlockSpec((BLK, DIM), lambda i: (i, 0)),
        ],
        out_shape=[
            jax.ShapeDtypeStruct((N_TOK, N_CODES), jnp.float32),
            jax.ShapeDtypeStruct((N_TOK,), jnp.int32),
            jax.ShapeDtypeStruct((N_TOK, DIM), jnp.float32),
        ],
    )(x, cbt2, codebook, c2)
    return (vec, codes, l2)
